# baseline (speedup 1.0000x reference)
import math

import jax
import jax.numpy as jnp
from jax import lax
from jax.experimental import pallas as pl
from jax.experimental.pallas import tpu as pltpu
from jax.experimental.pallas import tpu_sc as plsc

N = 10000
N_PAD = 10240
E = 320000
D = 128
D_OUT = 64
BN_INV = 1.0 / math.sqrt(1.0 + 1e-5)

NC = 2
NS = 16
NW = NC * NS
EPW = E // NW
CHUNK = 128
EPW_PAD = 10240
NGROUP = 2
GCHUNK = EPW_PAD // (NGROUP * CHUNK)
RPT = N_PAD // NS

RB = 2000


def _sc_agg_body(x_hbm, src_hbm, dst_hbm, zeros_hbm, out_hbm,
                 src_v, dst_v, rows_a, rows_b, acc_sh, sem_a, sem_b):
    c = lax.axis_index("c")
    s = lax.axis_index("s")
    wid = c * NS + s

    pltpu.sync_copy(zeros_hbm, rows_a)
    base = s * RPT

    def zcopy(t, carry):
        off = pl.multiple_of(base + t * CHUNK, 8)
        pltpu.sync_copy(rows_a, acc_sh.at[pl.ds(off, CHUNK)])
        return carry

    lax.fori_loop(0, RPT // CHUNK, zcopy, 0)
    plsc.subcore_barrier()

    def fire(j, buf, sem):
        pltpu.async_copy(x_hbm.at[src_v.at[j]], buf, sem)

    def drain(buf, sem):
        pltpu.make_async_copy(x_hbm.at[src_v.at[0]], buf, sem).wait()

    for g in range(NGROUP):
        pltpu.sync_copy(src_hbm.at[wid, g], src_v)
        pltpu.sync_copy(dst_hbm.at[wid, g], dst_v)

        fire(0, rows_a, sem_a)

        def step(i, carry):
            j = 2 * i
            fire(j + 1, rows_b, sem_b)
            drain(rows_a, sem_a)
            pltpu.sync_copy(rows_a, acc_sh.at[dst_v.at[j]], add=True)
            fire(j + 2, rows_a, sem_a)
            drain(rows_b, sem_b)
            pltpu.sync_copy(rows_b, acc_sh.at[dst_v.at[j + 1]], add=True)
            return carry

        lax.fori_loop(0, GCHUNK // 2 - 1, step, 0)
        fire(GCHUNK - 1, rows_b, sem_b)
        drain(rows_a, sem_a)
        pltpu.sync_copy(rows_a, acc_sh.at[dst_v.at[GCHUNK - 2]], add=True)
        drain(rows_b, sem_b)
        pltpu.sync_copy(rows_b, acc_sh.at[dst_v.at[GCHUNK - 1]], add=True)

    plsc.subcore_barrier()

    pltpu.sync_copy(acc_sh.at[pl.ds(base, RPT)], out_hbm.at[c, pl.ds(base, RPT)])


_sc_agg = pl.kernel(
    _sc_agg_body,
    out_type=jax.ShapeDtypeStruct((NC, N_PAD, D), jnp.float32),
    mesh=plsc.VectorSubcoreMesh(core_axis_name="c", subcore_axis_name="s",
                                num_cores=NC, num_subcores=NS),
    scratch_types=[
        pltpu.VMEM((GCHUNK, CHUNK), jnp.int32),
        pltpu.VMEM((GCHUNK, CHUNK), jnp.int32),
        pltpu.VMEM((CHUNK, D), jnp.float32),
        pltpu.VMEM((CHUNK, D), jnp.float32),
        pltpu.VMEM_SHARED((N_PAD, D), jnp.float32),
        pltpu.SemaphoreType.DMA,
        pltpu.SemaphoreType.DMA,
    ],
)


def _mlp1_body(x_ref, p_ref, W1a_ref, b1a_ref, g1_ref, be1_ref,
               W1b_ref, b1b_ref, o_ref):
    h = x_ref[...] + p_ref[0] + p_ref[1]
    h = jnp.dot(h, W1a_ref[...], preferred_element_type=jnp.float32) + b1a_ref[...]
    h = h * (g1_ref[...] * BN_INV) + be1_ref[...]
    h = jnp.maximum(h, 0.0)
    h = jnp.dot(h, W1b_ref[...], preferred_element_type=jnp.float32) + b1b_ref[...]
    o_ref[...] = jnp.maximum(h, 0.0)


def _mlp2_body(h_ref, q_ref, W2a_ref, b2a_ref, g2_ref, be2_ref,
               W2b_ref, b2b_ref, Wl1_ref, bl1_ref, Wl2_ref, bl2_ref, o_ref):
    h = h_ref[...] + q_ref[0] + q_ref[1]
    h = jnp.dot(h, W2a_ref[...], preferred_element_type=jnp.float32) + b2a_ref[...]
    h = h * (g2_ref[...] * BN_INV) + be2_ref[...]
    h = jnp.maximum(h, 0.0)
    h = jnp.dot(h, W2b_ref[...], preferred_element_type=jnp.float32) + b2b_ref[...]
    h = jnp.maximum(h, 0.0)
    h = jnp.dot(h, Wl1_ref[...], preferred_element_type=jnp.float32) + bl1_ref[...]
    h = jnp.maximum(h, 0.0)
    z = jnp.dot(h, Wl2_ref[...], preferred_element_type=jnp.float32) + bl2_ref[...]
    m = jnp.max(z, axis=1, keepdims=True)
    lse = jnp.log(jnp.sum(jnp.exp(z - m), axis=1, keepdims=True)) + m
    o_ref[...] = z - lse


def _row_block(shape_tail):
    return pl.BlockSpec((RB,) + shape_tail, lambda j: (j,) + (0,) * len(shape_tail))


def _partial_block():
    return pl.BlockSpec((NC, RB, D), lambda j: (0, j, 0))


def _full(shape):
    return pl.BlockSpec(shape, lambda j: (0,) * len(shape))


_mlp1 = pl.pallas_call(
    _mlp1_body,
    grid=(N // RB,),
    in_specs=[
        _row_block((D,)),
        _partial_block(),
        _full((D, D)), _full((1, D)), _full((1, D)), _full((1, D)),
        _full((D, D)), _full((1, D)),
    ],
    out_specs=_row_block((D,)),
    out_shape=jax.ShapeDtypeStruct((N, D), jnp.float32),
)

_mlp2 = pl.pallas_call(
    _mlp2_body,
    grid=(N // RB,),
    in_specs=[
        _row_block((D,)),
        _partial_block(),
        _full((D, D)), _full((1, D)), _full((1, D)), _full((1, D)),
        _full((D, D)), _full((1, D)),
        _full((D, D)), _full((1, D)),
        _full((D, D_OUT)), _full((1, D_OUT)),
    ],
    out_specs=_row_block((D_OUT,)),
    out_shape=jax.ShapeDtypeStruct((N, D_OUT), jnp.float32),
)


def kernel(x, edge_index, W1a, b1a, g1, be1, W1b, b1b,
           W2a, b2a, g2, be2, W2b, b2b, Wl1, bl1, Wl2, bl2):
    pad = EPW_PAD - EPW
    pad_src = jnp.broadcast_to(
        jnp.arange(pad, dtype=jnp.int32) * 37 % N, (NW, pad))
    src = jnp.concatenate([edge_index[0].reshape(NW, EPW), pad_src], axis=1)
    pad_dst = jnp.broadcast_to(N + jnp.arange(pad, dtype=jnp.int32), (NW, pad))
    dst = jnp.concatenate([edge_index[1].reshape(NW, EPW), pad_dst], axis=1)
    src = src.reshape(NW, NGROUP, GCHUNK, CHUNK)
    dst = dst.reshape(NW, NGROUP, GCHUNK, CHUNK)
    b1a, g1, be1, b1b = (v.reshape(1, D) for v in (b1a, g1, be1, b1b))
    b2a, g2, be2, b2b, bl1 = (v.reshape(1, D) for v in (b2a, g2, be2, b2b, bl1))
    bl2 = bl2.reshape(1, D_OUT)

    zrows = jnp.zeros((CHUNK, D), jnp.float32)
    p = _sc_agg(x, src, dst, zrows)
    h1 = _mlp1(x, p, W1a, b1a, g1, be1, W1b, b1b)
    q = _sc_agg(h1, src, dst, zrows)
    return _mlp2(h1, q, W2a, b2a, g2, be2, W2b, b2b, Wl1, bl1, Wl2, bl2)


# speedup vs baseline: 1.0196x; 1.0196x over previous
import math

import jax
import jax.numpy as jnp
from jax import lax
from jax.experimental import pallas as pl
from jax.experimental.pallas import tpu as pltpu
from jax.experimental.pallas import tpu_sc as plsc

N = 10000
N_PAD = 10240
E = 320000
D = 128
D_OUT = 64
BN_INV = 1.0 / math.sqrt(1.0 + 1e-5)

NC = 2
NS = 16
NW = NC * NS
EPW = E // NW
CHUNK = 128
EPW_PAD = 10240
NGROUP = 2
GCHUNK = EPW_PAD // (NGROUP * CHUNK)
RPT = N_PAD // NS

RB = 2000


def _sc_agg_body(x_hbm, src_hbm, dst_hbm, out_hbm,
                 src_v, dst_v, rows_a, rows_b, acc_sh, sem_a, sem_b):
    c = lax.axis_index("c")
    s = lax.axis_index("s")
    wid = c * NS + s

    zeros16 = jnp.zeros((16,), jnp.float32)

    def zrow(r, carry):
        def zcol(cc, carry2):
            rows_a[r, pl.ds(cc * 16, 16)] = zeros16
            return carry2
        return lax.fori_loop(0, D // 16, zcol, carry)

    lax.fori_loop(0, CHUNK, zrow, 0)
    base = s * RPT

    def zcopy(t, carry):
        off = pl.multiple_of(base + t * CHUNK, 8)
        pltpu.sync_copy(rows_a, acc_sh.at[pl.ds(off, CHUNK)])
        return carry

    lax.fori_loop(0, RPT // CHUNK, zcopy, 0)
    plsc.subcore_barrier()

    def fire(j, buf, sem):
        pltpu.async_copy(x_hbm.at[src_v.at[j]], buf, sem)

    def drain(buf, sem):
        pltpu.make_async_copy(x_hbm.at[src_v.at[0]], buf, sem).wait()

    for g in range(NGROUP):
        pltpu.sync_copy(src_hbm.at[wid, g], src_v)
        pltpu.sync_copy(dst_hbm.at[wid, g], dst_v)

        fire(0, rows_a, sem_a)

        def step(i, carry):
            j = 2 * i
            fire(j + 1, rows_b, sem_b)
            drain(rows_a, sem_a)
            pltpu.sync_copy(rows_a, acc_sh.at[dst_v.at[j]], add=True)
            fire(j + 2, rows_a, sem_a)
            drain(rows_b, sem_b)
            pltpu.sync_copy(rows_b, acc_sh.at[dst_v.at[j + 1]], add=True)
            return carry

        lax.fori_loop(0, GCHUNK // 2 - 1, step, 0)
        fire(GCHUNK - 1, rows_b, sem_b)
        drain(rows_a, sem_a)
        pltpu.sync_copy(rows_a, acc_sh.at[dst_v.at[GCHUNK - 2]], add=True)
        drain(rows_b, sem_b)
        pltpu.sync_copy(rows_b, acc_sh.at[dst_v.at[GCHUNK - 1]], add=True)

    plsc.subcore_barrier()

    pltpu.sync_copy(acc_sh.at[pl.ds(base, RPT)], out_hbm.at[c, pl.ds(base, RPT)])


_sc_agg = pl.kernel(
    _sc_agg_body,
    out_type=jax.ShapeDtypeStruct((NC, N_PAD, D), jnp.float32),
    mesh=plsc.VectorSubcoreMesh(core_axis_name="c", subcore_axis_name="s",
                                num_cores=NC, num_subcores=NS),
    scratch_types=[
        pltpu.VMEM((GCHUNK, CHUNK), jnp.int32),
        pltpu.VMEM((GCHUNK, CHUNK), jnp.int32),
        pltpu.VMEM((CHUNK, D), jnp.float32),
        pltpu.VMEM((CHUNK, D), jnp.float32),
        pltpu.VMEM_SHARED((N_PAD, D), jnp.float32),
        pltpu.SemaphoreType.DMA,
        pltpu.SemaphoreType.DMA,
    ],
)


def _mlp1_body(x_ref, p_ref, W1a_ref, b1a_ref, g1_ref, be1_ref,
               W1b_ref, b1b_ref, o_ref):
    h = x_ref[...] + p_ref[0] + p_ref[1]
    h = jnp.dot(h, W1a_ref[...], preferred_element_type=jnp.float32) + b1a_ref[...]
    h = h * (g1_ref[...] * BN_INV) + be1_ref[...]
    h = jnp.maximum(h, 0.0)
    h = jnp.dot(h, W1b_ref[...], preferred_element_type=jnp.float32) + b1b_ref[...]
    o_ref[...] = jnp.maximum(h, 0.0)


def _mlp2_body(h_ref, q_ref, W2a_ref, b2a_ref, g2_ref, be2_ref,
               W2b_ref, b2b_ref, Wl1_ref, bl1_ref, Wl2_ref, bl2_ref, o_ref):
    h = h_ref[...] + q_ref[0] + q_ref[1]
    h = jnp.dot(h, W2a_ref[...], preferred_element_type=jnp.float32) + b2a_ref[...]
    h = h * (g2_ref[...] * BN_INV) + be2_ref[...]
    h = jnp.maximum(h, 0.0)
    h = jnp.dot(h, W2b_ref[...], preferred_element_type=jnp.float32) + b2b_ref[...]
    h = jnp.maximum(h, 0.0)
    h = jnp.dot(h, Wl1_ref[...], preferred_element_type=jnp.float32) + bl1_ref[...]
    h = jnp.maximum(h, 0.0)
    z = jnp.dot(h, Wl2_ref[...], preferred_element_type=jnp.float32) + bl2_ref[...]
    m = jnp.max(z, axis=1, keepdims=True)
    lse = jnp.log(jnp.sum(jnp.exp(z - m), axis=1, keepdims=True)) + m
    o_ref[...] = z - lse


def _row_block(shape_tail):
    return pl.BlockSpec((RB,) + shape_tail, lambda j: (j,) + (0,) * len(shape_tail))


def _partial_block():
    return pl.BlockSpec((NC, RB, D), lambda j: (0, j, 0))


def _full(shape):
    return pl.BlockSpec(shape, lambda j: (0,) * len(shape))


_mlp1 = pl.pallas_call(
    _mlp1_body,
    grid=(N // RB,),
    in_specs=[
        _row_block((D,)),
        _partial_block(),
        _full((D, D)), _full((1, D)), _full((1, D)), _full((1, D)),
        _full((D, D)), _full((1, D)),
    ],
    out_specs=_row_block((D,)),
    out_shape=jax.ShapeDtypeStruct((N, D), jnp.float32),
)

_mlp2 = pl.pallas_call(
    _mlp2_body,
    grid=(N // RB,),
    in_specs=[
        _row_block((D,)),
        _partial_block(),
        _full((D, D)), _full((1, D)), _full((1, D)), _full((1, D)),
        _full((D, D)), _full((1, D)),
        _full((D, D)), _full((1, D)),
        _full((D, D_OUT)), _full((1, D_OUT)),
    ],
    out_specs=_row_block((D_OUT,)),
    out_shape=jax.ShapeDtypeStruct((N, D_OUT), jnp.float32),
)


def kernel(x, edge_index, W1a, b1a, g1, be1, W1b, b1b,
           W2a, b2a, g2, be2, W2b, b2b, Wl1, bl1, Wl2, bl2):
    pad = EPW_PAD - EPW
    pad_src = jnp.broadcast_to(
        jnp.arange(pad, dtype=jnp.int32) * 37 % N, (NW, pad))
    src = jnp.concatenate([edge_index[0].reshape(NW, EPW), pad_src], axis=1)
    pad_dst = jnp.broadcast_to(N + jnp.arange(pad, dtype=jnp.int32), (NW, pad))
    dst = jnp.concatenate([edge_index[1].reshape(NW, EPW), pad_dst], axis=1)
    src = src.reshape(NW, NGROUP, GCHUNK, CHUNK)
    dst = dst.reshape(NW, NGROUP, GCHUNK, CHUNK)
    b1a, g1, be1, b1b = (v.reshape(1, D) for v in (b1a, g1, be1, b1b))
    b2a, g2, be2, b2b, bl1 = (v.reshape(1, D) for v in (b2a, g2, be2, b2b, bl1))
    bl2 = bl2.reshape(1, D_OUT)

    p = _sc_agg(x, src, dst)
    h1 = _mlp1(x, p, W1a, b1a, g1, be1, W1b, b1b)
    q = _sc_agg(h1, src, dst)
    return _mlp2(h1, q, W2a, b2a, g2, be2, W2b, b2b, Wl1, bl1, Wl2, bl2)


# speedup vs baseline: 1.0203x; 1.0006x over previous
import math

import jax
import jax.numpy as jnp
from jax import lax
from jax.experimental import pallas as pl
from jax.experimental.pallas import tpu as pltpu
from jax.experimental.pallas import tpu_sc as plsc

N = 10000
N_PAD = 10240
E = 320000
D = 128
D_OUT = 64
BN_INV = 1.0 / math.sqrt(1.0 + 1e-5)

NC = 2
NS = 16
NW = NC * NS
EPW = E // NW
CHUNK = 128
EPW_PAD = 10240
NGROUP = 2
GCHUNK = EPW_PAD // (NGROUP * CHUNK)
RPT = N_PAD // NS

RB = 2000


def _sc_agg_body(x_hbm, src_hbm, dst_hbm, out_hbm,
                 src_v, dst_v, rows_a, rows_b, acc_sh, sem_a, sem_b):
    c = lax.axis_index("c")
    s = lax.axis_index("s")
    wid = c * NS + s

    zeros16 = jnp.zeros((16,), jnp.float32)

    def zrow(r, carry):
        def zcol(cc, carry2):
            rows_a[r, pl.ds(cc * 16, 16)] = zeros16
            return carry2
        return lax.fori_loop(0, D // 16, zcol, carry)

    lax.fori_loop(0, CHUNK, zrow, 0)
    base = s * RPT

    def zcopy(t, carry):
        off = pl.multiple_of(base + t * CHUNK, 8)
        pltpu.sync_copy(rows_a, acc_sh.at[pl.ds(off, CHUNK)])
        return carry

    lax.fori_loop(0, RPT // CHUNK, zcopy, 0)
    plsc.subcore_barrier()

    def fire(j, buf, sem):
        pltpu.async_copy(x_hbm.at[src_v.at[j]], buf, sem)

    def drain(buf, sem):
        pltpu.make_async_copy(x_hbm.at[src_v.at[0]], buf, sem).wait()

    for g in range(NGROUP):
        pltpu.sync_copy(src_hbm.at[wid, g], src_v)
        pltpu.sync_copy(dst_hbm.at[wid, g], dst_v)

        fire(0, rows_a, sem_a)

        def step(i, carry):
            j = 2 * i
            fire(j + 1, rows_b, sem_b)
            drain(rows_a, sem_a)
            pltpu.sync_copy(rows_a, acc_sh.at[dst_v.at[j]], add=True)
            fire(j + 2, rows_a, sem_a)
            drain(rows_b, sem_b)
            pltpu.sync_copy(rows_b, acc_sh.at[dst_v.at[j + 1]], add=True)
            return carry

        lax.fori_loop(0, GCHUNK // 2 - 1, step, 0)
        fire(GCHUNK - 1, rows_b, sem_b)
        drain(rows_a, sem_a)
        pltpu.sync_copy(rows_a, acc_sh.at[dst_v.at[GCHUNK - 2]], add=True)
        drain(rows_b, sem_b)
        pltpu.sync_copy(rows_b, acc_sh.at[dst_v.at[GCHUNK - 1]], add=True)

    plsc.subcore_barrier()

    pltpu.sync_copy(acc_sh.at[pl.ds(base, RPT)], out_hbm.at[c, pl.ds(base, RPT)])


_sc_agg = pl.kernel(
    _sc_agg_body,
    out_type=jax.ShapeDtypeStruct((NC, N_PAD, D), jnp.float32),
    mesh=plsc.VectorSubcoreMesh(core_axis_name="c", subcore_axis_name="s",
                                num_cores=NC, num_subcores=NS),
    scratch_types=[
        pltpu.VMEM((GCHUNK, CHUNK), jnp.int32),
        pltpu.VMEM((GCHUNK, CHUNK), jnp.int32),
        pltpu.VMEM((CHUNK, D), jnp.float32),
        pltpu.VMEM((CHUNK, D), jnp.float32),
        pltpu.VMEM_SHARED((N_PAD, D), jnp.float32),
        pltpu.SemaphoreType.DMA,
        pltpu.SemaphoreType.DMA,
    ],
)


def _mlp1_body(x_ref, p_ref, W1a_ref, b1a_ref, g1_ref, be1_ref,
               W1b_ref, b1b_ref, o_ref):
    h = x_ref[...] + p_ref[0, :N] + p_ref[1, :N]
    h = jnp.dot(h, W1a_ref[...], preferred_element_type=jnp.float32) + b1a_ref[...]
    h = h * (g1_ref[...] * BN_INV) + be1_ref[...]
    h = jnp.maximum(h, 0.0)
    h = jnp.dot(h, W1b_ref[...], preferred_element_type=jnp.float32) + b1b_ref[...]
    o_ref[...] = jnp.maximum(h, 0.0)


def _mlp2_body(h_ref, q_ref, W2a_ref, b2a_ref, g2_ref, be2_ref,
               W2b_ref, b2b_ref, Wl1_ref, bl1_ref, Wl2_ref, bl2_ref, o_ref):
    h = h_ref[...] + q_ref[0, :N] + q_ref[1, :N]
    h = jnp.dot(h, W2a_ref[...], preferred_element_type=jnp.float32) + b2a_ref[...]
    h = h * (g2_ref[...] * BN_INV) + be2_ref[...]
    h = jnp.maximum(h, 0.0)
    h = jnp.dot(h, W2b_ref[...], preferred_element_type=jnp.float32) + b2b_ref[...]
    h = jnp.maximum(h, 0.0)
    h = jnp.dot(h, Wl1_ref[...], preferred_element_type=jnp.float32) + bl1_ref[...]
    h = jnp.maximum(h, 0.0)
    z = jnp.dot(h, Wl2_ref[...], preferred_element_type=jnp.float32) + bl2_ref[...]
    m = jnp.max(z, axis=1, keepdims=True)
    lse = jnp.log(jnp.sum(jnp.exp(z - m), axis=1, keepdims=True)) + m
    o_ref[...] = z - lse


def _row_block(shape_tail):
    return pl.BlockSpec((RB,) + shape_tail, lambda j: (j,) + (0,) * len(shape_tail))


def _partial_block():
    return pl.BlockSpec((NC, RB, D), lambda j: (0, j, 0))


def _full(shape):
    return pl.BlockSpec(shape, lambda j: (0,) * len(shape))


_mlp1 = pl.pallas_call(
    _mlp1_body,
    out_shape=jax.ShapeDtypeStruct((N, D), jnp.float32),
)

_mlp2 = pl.pallas_call(
    _mlp2_body,
    out_shape=jax.ShapeDtypeStruct((N, D_OUT), jnp.float32),
)


def kernel(x, edge_index, W1a, b1a, g1, be1, W1b, b1b,
           W2a, b2a, g2, be2, W2b, b2b, Wl1, bl1, Wl2, bl2):
    pad = EPW_PAD - EPW
    pad_src = jnp.broadcast_to(
        jnp.arange(pad, dtype=jnp.int32) * 37 % N, (NW, pad))
    src = jnp.concatenate([edge_index[0].reshape(NW, EPW), pad_src], axis=1)
    pad_dst = jnp.broadcast_to(N + jnp.arange(pad, dtype=jnp.int32), (NW, pad))
    dst = jnp.concatenate([edge_index[1].reshape(NW, EPW), pad_dst], axis=1)
    src = src.reshape(NW, NGROUP, GCHUNK, CHUNK)
    dst = dst.reshape(NW, NGROUP, GCHUNK, CHUNK)
    b1a, g1, be1, b1b = (v.reshape(1, D) for v in (b1a, g1, be1, b1b))
    b2a, g2, be2, b2b, bl1 = (v.reshape(1, D) for v in (b2a, g2, be2, b2b, bl1))
    bl2 = bl2.reshape(1, D_OUT)

    p = _sc_agg(x, src, dst)
    h1 = _mlp1(x, p, W1a, b1a, g1, be1, W1b, b1b)
    q = _sc_agg(h1, src, dst)
    return _mlp2(h1, q, W2a, b2a, g2, be2, W2b, b2b, Wl1, bl1, Wl2, bl2)


# speedup vs baseline: 1.0208x; 1.0005x over previous
import math

import jax
import jax.numpy as jnp
from jax import lax
from jax.experimental import pallas as pl
from jax.experimental.pallas import tpu as pltpu
from jax.experimental.pallas import tpu_sc as plsc

N = 10000
N_PAD = 10240
E = 320000
D = 128
D_OUT = 64
BN_INV = 1.0 / math.sqrt(1.0 + 1e-5)

NC = 2
NS = 16
NW = NC * NS
EPW = E // NW
CHUNK = 128
EPW_PAD = 10240
NGROUP = 2
GCHUNK = EPW_PAD // (NGROUP * CHUNK)
RPT = N_PAD // NS


def _sc_agg_body(x_hbm, src_hbm, dst_hbm, out_hbm,
                 src_v, dst_v, rows_a, rows_b, acc_sh, sem_a, sem_b):
    c = lax.axis_index("c")
    s = lax.axis_index("s")
    wid = c * NS + s

    zeros16 = jnp.zeros((16,), jnp.float32)

    def zrow(r, carry):
        def zcol(cc, carry2):
            rows_a[r, pl.ds(cc * 16, 16)] = zeros16
            return carry2
        return lax.fori_loop(0, D // 16, zcol, carry)

    lax.fori_loop(0, CHUNK, zrow, 0)
    base = s * RPT

    def zcopy(t, carry):
        off = pl.multiple_of(base + t * CHUNK, 8)
        pltpu.sync_copy(rows_a, acc_sh.at[pl.ds(off, CHUNK)])
        return carry

    lax.fori_loop(0, RPT // CHUNK, zcopy, 0)
    plsc.subcore_barrier()

    def fire(j, buf, sem):
        pltpu.async_copy(x_hbm.at[src_v.at[j]], buf, sem)

    def drain(buf, sem):
        pltpu.make_async_copy(x_hbm.at[src_v.at[0]], buf, sem).wait()

    for g in range(NGROUP):
        pltpu.sync_copy(src_hbm.at[wid, g], src_v)
        pltpu.sync_copy(dst_hbm.at[wid, g], dst_v)

        fire(0, rows_a, sem_a)

        def step(i, carry):
            j = 2 * i
            fire(j + 1, rows_b, sem_b)
            drain(rows_a, sem_a)
            pltpu.sync_copy(rows_a, acc_sh.at[dst_v.at[j]], add=True)
            fire(j + 2, rows_a, sem_a)
            drain(rows_b, sem_b)
            pltpu.sync_copy(rows_b, acc_sh.at[dst_v.at[j + 1]], add=True)
            return carry

        lax.fori_loop(0, GCHUNK // 2 - 1, step, 0)
        fire(GCHUNK - 1, rows_b, sem_b)
        drain(rows_a, sem_a)
        pltpu.sync_copy(rows_a, acc_sh.at[dst_v.at[GCHUNK - 2]], add=True)
        drain(rows_b, sem_b)
        pltpu.sync_copy(rows_b, acc_sh.at[dst_v.at[GCHUNK - 1]], add=True)

    plsc.subcore_barrier()

    pltpu.sync_copy(acc_sh.at[pl.ds(base, RPT)], out_hbm.at[c, pl.ds(base, RPT)])


_sc_agg = pl.kernel(
    _sc_agg_body,
    out_type=jax.ShapeDtypeStruct((NC, N_PAD, D), jnp.float32),
    mesh=plsc.VectorSubcoreMesh(core_axis_name="c", subcore_axis_name="s",
                                num_cores=NC, num_subcores=NS),
    scratch_types=[
        pltpu.VMEM((GCHUNK, CHUNK), jnp.int32),
        pltpu.VMEM((GCHUNK, CHUNK), jnp.int32),
        pltpu.VMEM((CHUNK, D), jnp.float32),
        pltpu.VMEM((CHUNK, D), jnp.float32),
        pltpu.VMEM_SHARED((N_PAD, D), jnp.float32),
        pltpu.SemaphoreType.DMA,
        pltpu.SemaphoreType.DMA,
    ],
)


def _mlp1_body(x_ref, p_ref, W1a_ref, b1a_ref, g1_ref, be1_ref,
               W1b_ref, b1b_ref, o_ref):
    h = x_ref[...] + p_ref[0, :N] + p_ref[1, :N]
    h = jnp.dot(h, W1a_ref[...], preferred_element_type=jnp.float32) + b1a_ref[...]
    h = h * (g1_ref[...] * BN_INV) + be1_ref[...]
    h = jnp.maximum(h, 0.0)
    h = jnp.dot(h, W1b_ref[...], preferred_element_type=jnp.float32) + b1b_ref[...]
    o_ref[...] = jnp.maximum(h, 0.0)


def _mlp2_body(h_ref, q_ref, W2a_ref, b2a_ref, g2_ref, be2_ref,
               W2b_ref, b2b_ref, Wl1_ref, bl1_ref, Wl2_ref, bl2_ref, o_ref):
    h = h_ref[...] + q_ref[0, :N] + q_ref[1, :N]
    h = jnp.dot(h, W2a_ref[...], preferred_element_type=jnp.float32) + b2a_ref[...]
    h = h * (g2_ref[...] * BN_INV) + be2_ref[...]
    h = jnp.maximum(h, 0.0)
    h = jnp.dot(h, W2b_ref[...], preferred_element_type=jnp.float32) + b2b_ref[...]
    h = jnp.maximum(h, 0.0)
    h = jnp.dot(h, Wl1_ref[...], preferred_element_type=jnp.float32) + bl1_ref[...]
    h = jnp.maximum(h, 0.0)
    z = jnp.dot(h, Wl2_ref[...], preferred_element_type=jnp.float32) + bl2_ref[...]
    m = jnp.max(z, axis=1, keepdims=True)
    lse = jnp.log(jnp.sum(jnp.exp(z - m), axis=1, keepdims=True)) + m
    o_ref[...] = z - lse


_mlp1 = pl.pallas_call(
    _mlp1_body,
    out_shape=jax.ShapeDtypeStruct((N, D), jnp.float32),
)

_mlp2 = pl.pallas_call(
    _mlp2_body,
    out_shape=jax.ShapeDtypeStruct((N, D_OUT), jnp.float32),
)


def kernel(x, edge_index, W1a, b1a, g1, be1, W1b, b1b,
           W2a, b2a, g2, be2, W2b, b2b, Wl1, bl1, Wl2, bl2):
    pad = EPW_PAD - EPW
    pad_src = jnp.broadcast_to(
        jnp.arange(pad, dtype=jnp.int32) * 37 % N, (NW, pad))
    src = jnp.concatenate([edge_index[0].reshape(NW, EPW), pad_src], axis=1)
    pad_dst = jnp.broadcast_to(N + jnp.arange(pad, dtype=jnp.int32), (NW, pad))
    dst = jnp.concatenate([edge_index[1].reshape(NW, EPW), pad_dst], axis=1)
    src = src.reshape(NW, NGROUP, GCHUNK, CHUNK)
    dst = dst.reshape(NW, NGROUP, GCHUNK, CHUNK)
    b1a, g1, be1, b1b = (v.reshape(1, D) for v in (b1a, g1, be1, b1b))
    b2a, g2, be2, b2b, bl1 = (v.reshape(1, D) for v in (b2a, g2, be2, b2b, bl1))
    bl2 = bl2.reshape(1, D_OUT)

    p = _sc_agg(x, src, dst)
    h1 = _mlp1(x, p, W1a, b1a, g1, be1, W1b, b1b)
    q = _sc_agg(h1, src, dst)
    return _mlp2(h1, q, W2a, b2a, g2, be2, W2b, b2b, Wl1, bl1, Wl2, bl2)
